# Initial kernel scaffold; baseline (speedup 1.0000x reference)
import jax
import jax.numpy as jnp
from jax.experimental import pallas as pl


def kernel(x, edge_index, batch, W1, b1, W2, b2, eps, gamma, beta, vW1, vb1, vW2, vb2, pW, pb):
    raise NotImplementedError("write your pallas kernel here")


# speedup vs baseline: 1.0967x; 1.0967x over previous
import functools

import jax
import jax.numpy as jnp
from jax import lax
from jax.experimental import pallas as pl
from jax.experimental.pallas import tpu as pltpu

_N = 10000
_E = 320000
_D = 128
_G = 128
_L = 5
_BR = 400
_NB = _N // _BR

_INTERPRET = False


def _mlp_seg_body(hp_ref, agg_ref, b3_ref, w1_ref, b1_ref, w2_ref, b2_ref,
                  gm_ref, bt_ref, eps_ref, h2_ref, seg_ref, *rest,
                  relu_out, with_cnt):
    i = pl.program_id(0)
    e = eps_ref[0, 0]
    m = (1.0 + e) * hp_ref[...] + jnp.sum(agg_ref[...], axis=0)
    a = jnp.maximum(
        jnp.dot(m, w1_ref[...], preferred_element_type=jnp.float32)
        + b1_ref[...], 0.0)
    o = jnp.dot(a, w2_ref[...], preferred_element_type=jnp.float32) + b2_ref[...]
    o = gm_ref[...] * o + bt_ref[...]
    if relu_out:
        o = jnp.maximum(o, 0.0)
    h2_ref[...] = o
    b = b3_ref[0, 0, :]
    oh = (b[:, None] == lax.broadcasted_iota(jnp.int32, (_BR, _G), 1)
          ).astype(jnp.float32)
    contrib = lax.dot_general(oh, o, (((0,), (0,)), ((), ())),
                              preferred_element_type=jnp.float32)

    @pl.when(i == 0)
    def _():
        seg_ref[...] = jnp.zeros_like(seg_ref)

    seg_ref[...] += contrib
    if with_cnt:
        cnt_ref = rest[0]

        @pl.when(i == 0)
        def _():
            cnt_ref[...] = jnp.zeros_like(cnt_ref)

        cnt_ref[...] += jnp.sum(oh, axis=0, keepdims=True)


def _mlp_seg(hp, agg2, batch3, w1, b1, w2, b2, gm, bt, eps1, relu_out, with_cnt):
    p = agg2.shape[0]
    out_shapes = [
        jax.ShapeDtypeStruct((_N, _D), jnp.float32),
        jax.ShapeDtypeStruct((_G, _D), jnp.float32),
    ]
    out_specs = [
        pl.BlockSpec((_BR, _D), lambda i: (i, 0)),
        pl.BlockSpec((_G, _D), lambda i: (0, 0)),
    ]
    if with_cnt:
        out_shapes.append(jax.ShapeDtypeStruct((1, _G), jnp.float32))
        out_specs.append(pl.BlockSpec((1, _G), lambda i: (0, 0)))
    return pl.pallas_call(
        functools.partial(_mlp_seg_body, relu_out=relu_out, with_cnt=with_cnt),
        grid=(_NB,),
        in_specs=[
            pl.BlockSpec((_BR, _D), lambda i: (i, 0)),
            pl.BlockSpec((p, _BR, _D), lambda i: (0, i, 0)),
            pl.BlockSpec((1, 1, _BR), lambda i: (i, 0, 0)),
            pl.BlockSpec((_D, 2 * _D), lambda i: (0, 0)),
            pl.BlockSpec((1, 2 * _D), lambda i: (0, 0)),
            pl.BlockSpec((2 * _D, _D), lambda i: (0, 0)),
            pl.BlockSpec((1, _D), lambda i: (0, 0)),
            pl.BlockSpec((1, _D), lambda i: (0, 0)),
            pl.BlockSpec((1, _D), lambda i: (0, 0)),
            pl.BlockSpec((1, 1), lambda i: (0, 0)),
        ],
        out_specs=out_specs,
        out_shape=out_shapes,
        interpret=_INTERPRET,
    )(hp, agg2, batch3, w1, b1, w2, b2, gm, bt, eps1)


def _vn_body(h2_ref, seg_ref, vn_ref, vw1_ref, vb1_ref, vw2_ref, vb2_ref,
             b3_ref, hp_ref, vno_ref, vns_ref):
    i = pl.program_id(0)

    @pl.when(i == 0)
    def _():
        vt = seg_ref[...] + vn_ref[...]
        a = jnp.maximum(
            jnp.dot(vt, vw1_ref[...], preferred_element_type=jnp.float32)
            + vb1_ref[...], 0.0)
        v2 = jnp.maximum(
            jnp.dot(a, vw2_ref[...], preferred_element_type=jnp.float32)
            + vb2_ref[...], 0.0)
        vns_ref[...] = v2

    vno_ref[...] = vns_ref[...]
    b = b3_ref[0, 0, :]
    oh = (b[:, None] == lax.broadcasted_iota(jnp.int32, (_BR, _G), 1)
          ).astype(jnp.float32)
    hp_ref[...] = h2_ref[...] + jnp.dot(oh, vns_ref[...],
                                        preferred_element_type=jnp.float32)


def _vn_update(h2, seg, vn, vw1, vb1, vw2, vb2, batch3):
    return pl.pallas_call(
        _vn_body,
        grid=(_NB,),
        in_specs=[
            pl.BlockSpec((_BR, _D), lambda i: (i, 0)),
            pl.BlockSpec((_G, _D), lambda i: (0, 0)),
            pl.BlockSpec((_G, _D), lambda i: (0, 0)),
            pl.BlockSpec((_D, 2 * _D), lambda i: (0, 0)),
            pl.BlockSpec((1, 2 * _D), lambda i: (0, 0)),
            pl.BlockSpec((2 * _D, _D), lambda i: (0, 0)),
            pl.BlockSpec((1, _D), lambda i: (0, 0)),
            pl.BlockSpec((1, 1, _BR), lambda i: (i, 0, 0)),
        ],
        out_specs=[
            pl.BlockSpec((_BR, _D), lambda i: (i, 0)),
            pl.BlockSpec((_G, _D), lambda i: (0, 0)),
        ],
        out_shape=[
            jax.ShapeDtypeStruct((_N, _D), jnp.float32),
            jax.ShapeDtypeStruct((_G, _D), jnp.float32),
        ],
        scratch_shapes=[pltpu.VMEM((_G, _D), jnp.float32)],
        interpret=_INTERPRET,
    )(h2, seg, vn, vw1, vb1, vw2, vb2, batch3)


def _head_body(seg_ref, cnt_ref, pw_ref, pb_ref, out_ref):
    c = jnp.maximum(cnt_ref[0, :], 1.0)
    hg = seg_ref[...] * (1.0 / c)[:, None]
    out_ref[...] = jnp.dot(hg, pw_ref[...],
                           preferred_element_type=jnp.float32) + pb_ref[...]


def _head(seg, cnt, pw, pb1, t):
    return pl.pallas_call(
        _head_body,
        out_shape=jax.ShapeDtypeStruct((_G, t), jnp.float32),
        interpret=_INTERPRET,
    )(seg, cnt, pw, pb1)


def kernel(x, edge_index, batch, W1, b1, W2, b2, eps, gamma, beta,
           vW1, vb1, vW2, vb2, pW, pb):
    src = edge_index[0]
    dst = edge_index[1]
    t = pW.shape[1]
    batch3 = batch.reshape(_NB, 1, _BR)
    eps1 = eps.reshape(_L, 1, 1)
    b1r = b1.reshape(_L, 1, 2 * _D)
    b2r = b2.reshape(_L, 1, _D)
    gmr = gamma.reshape(_L, 1, _D)
    btr = beta.reshape(_L, 1, _D)
    vb1r = vb1.reshape(_L - 1, 1, 2 * _D)
    vb2r = vb2.reshape(_L - 1, 1, _D)
    pb1 = pb.reshape(1, t)

    vn = jnp.zeros((_G, _D), jnp.float32)
    hp = x
    seg = None
    cnt = None
    for l in range(_L):
        agg = jnp.zeros_like(hp).at[dst].add(hp[src])
        agg2 = agg[None]
        last = l == _L - 1
        outs = _mlp_seg(hp, agg2, batch3, W1[l], b1r[l], W2[l], b2r[l],
                        gmr[l], btr[l], eps1[l], relu_out=not last,
                        with_cnt=last)
        if last:
            h2, seg, cnt = outs
        else:
            h2, seg = outs
            hp, vn = _vn_update(h2, seg, vn, vW1[l], vb1r[l], vW2[l],
                                vb2r[l], batch3)
    return _head(seg, cnt, pW, pb1, t)


# speedup vs baseline: 7.1007x; 6.4747x over previous
import functools

import jax
import jax.numpy as jnp
from jax import lax
from jax.experimental import pallas as pl
from jax.experimental.pallas import tpu as pltpu
from jax.experimental.pallas import tpu_sc as plsc

_N = 10000
_E = 320000
_D = 128
_G = 128
_L = 5
_NP = 10240
_BR = 512
_NB = _NP // _BR

_INTERPRET = False

_NC = 2
_NS = 16
_NW = _NC * _NS
_EPW = _E // _NW
_C = 125
_NCH = _EPW // _C
_RPT = _NP // _NS


def _sc_agg_body(hp_hbm, src_hbm, dst_hbm, zer_hbm, out_hbm,
                 sidx_v, didx_v, rows0_v, acc_sh, gsem):
    cid = lax.axis_index("c")
    sid = lax.axis_index("s")
    wid = sid * _NC + cid
    pltpu.sync_copy(src_hbm.at[wid], sidx_v)
    pltpu.sync_copy(dst_hbm.at[wid], didx_v)
    pltpu.sync_copy(zer_hbm, acc_sh.at[pl.ds(sid * _RPT, _RPT)])
    plsc.subcore_barrier()

    def body(k, carry):
        pltpu.async_copy(hp_hbm.at[sidx_v.at[k]], rows0_v, gsem).wait()
        pltpu.sync_copy(rows0_v, acc_sh.at[didx_v.at[k]], add=True)
        return carry

    lax.fori_loop(0, _NCH, body, 0, unroll=False)
    plsc.subcore_barrier()
    pltpu.sync_copy(acc_sh.at[pl.ds(sid * _RPT, _RPT)],
                    out_hbm.at[cid, pl.ds(sid * _RPT, _RPT)])


def _sc_agg(hp, src3, dst3, zer):
    mesh = plsc.VectorSubcoreMesh(core_axis_name="c", subcore_axis_name="s",
                                  num_cores=_NC, num_subcores=_NS)
    f = pl.kernel(
        _sc_agg_body,
        out_type=jax.ShapeDtypeStruct((_NC, _NP, _D), jnp.float32),
        mesh=mesh,
        scratch_types=[
            pltpu.VMEM((_NCH, _C), jnp.int32),
            pltpu.VMEM((_NCH, _C), jnp.int32),
            pltpu.VMEM((_C, _D), jnp.float32),
            pltpu.VMEM_SHARED((_NP, _D), jnp.float32),
            pltpu.SemaphoreType.DMA,
        ],
    )
    return f(hp, src3, dst3, zer)


def _mlp_seg_body(hp_ref, agg_ref, b3_ref, w1_ref, b1_ref, w2_ref, b2_ref,
                  gm_ref, bt_ref, eps_ref, h2_ref, seg_ref, *rest,
                  relu_out, with_cnt):
    i = pl.program_id(0)
    e = eps_ref[0, 0]
    m = (1.0 + e) * hp_ref[...] + jnp.sum(agg_ref[...], axis=0)
    a = jnp.maximum(
        jnp.dot(m, w1_ref[...], preferred_element_type=jnp.float32)
        + b1_ref[...], 0.0)
    o = jnp.dot(a, w2_ref[...], preferred_element_type=jnp.float32) + b2_ref[...]
    o = gm_ref[...] * o + bt_ref[...]
    if relu_out:
        o = jnp.maximum(o, 0.0)
    h2_ref[...] = o
    b = b3_ref[0, 0, :]
    oh = (b[:, None] == lax.broadcasted_iota(jnp.int32, (_BR, _G), 1)
          ).astype(jnp.float32)
    contrib = lax.dot_general(oh, o, (((0,), (0,)), ((), ())),
                              preferred_element_type=jnp.float32)

    @pl.when(i == 0)
    def _():
        seg_ref[...] = jnp.zeros_like(seg_ref)

    seg_ref[...] += contrib
    if with_cnt:
        cnt_ref = rest[0]

        @pl.when(i == 0)
        def _():
            cnt_ref[...] = jnp.zeros_like(cnt_ref)

        cnt_ref[...] += jnp.sum(oh, axis=0, keepdims=True)


def _mlp_seg(hp, agg2, batch3, w1, b1, w2, b2, gm, bt, eps1, relu_out, with_cnt):
    p = agg2.shape[0]
    out_shapes = [
        jax.ShapeDtypeStruct((_NP, _D), jnp.float32),
        jax.ShapeDtypeStruct((_G, _D), jnp.float32),
    ]
    out_specs = [
        pl.BlockSpec((_BR, _D), lambda i: (i, 0)),
        pl.BlockSpec((_G, _D), lambda i: (0, 0)),
    ]
    if with_cnt:
        out_shapes.append(jax.ShapeDtypeStruct((1, _G), jnp.float32))
        out_specs.append(pl.BlockSpec((1, _G), lambda i: (0, 0)))
    return pl.pallas_call(
        functools.partial(_mlp_seg_body, relu_out=relu_out, with_cnt=with_cnt),
        grid=(_NB,),
        in_specs=[
            pl.BlockSpec((_BR, _D), lambda i: (i, 0)),
            pl.BlockSpec((p, _BR, _D), lambda i: (0, i, 0)),
            pl.BlockSpec((1, 1, _BR), lambda i: (i, 0, 0)),
            pl.BlockSpec((_D, 2 * _D), lambda i: (0, 0)),
            pl.BlockSpec((1, 2 * _D), lambda i: (0, 0)),
            pl.BlockSpec((2 * _D, _D), lambda i: (0, 0)),
            pl.BlockSpec((1, _D), lambda i: (0, 0)),
            pl.BlockSpec((1, _D), lambda i: (0, 0)),
            pl.BlockSpec((1, _D), lambda i: (0, 0)),
            pl.BlockSpec((1, 1), lambda i: (0, 0)),
        ],
        out_specs=out_specs,
        out_shape=out_shapes,
        interpret=_INTERPRET,
    )(hp, agg2, batch3, w1, b1, w2, b2, gm, bt, eps1)


def _vn_body(h2_ref, seg_ref, vn_ref, vw1_ref, vb1_ref, vw2_ref, vb2_ref,
             b3_ref, hp_ref, vno_ref, vns_ref):
    i = pl.program_id(0)

    @pl.when(i == 0)
    def _():
        vt = seg_ref[...] + vn_ref[...]
        a = jnp.maximum(
            jnp.dot(vt, vw1_ref[...], preferred_element_type=jnp.float32)
            + vb1_ref[...], 0.0)
        v2 = jnp.maximum(
            jnp.dot(a, vw2_ref[...], preferred_element_type=jnp.float32)
            + vb2_ref[...], 0.0)
        vns_ref[...] = v2

    vno_ref[...] = vns_ref[...]
    b = b3_ref[0, 0, :]
    oh = (b[:, None] == lax.broadcasted_iota(jnp.int32, (_BR, _G), 1)
          ).astype(jnp.float32)
    hp_ref[...] = h2_ref[...] + jnp.dot(oh, vns_ref[...],
                                        preferred_element_type=jnp.float32)


def _vn_update(h2, seg, vn, vw1, vb1, vw2, vb2, batch3):
    return pl.pallas_call(
        _vn_body,
        grid=(_NB,),
        in_specs=[
            pl.BlockSpec((_BR, _D), lambda i: (i, 0)),
            pl.BlockSpec((_G, _D), lambda i: (0, 0)),
            pl.BlockSpec((_G, _D), lambda i: (0, 0)),
            pl.BlockSpec((_D, 2 * _D), lambda i: (0, 0)),
            pl.BlockSpec((1, 2 * _D), lambda i: (0, 0)),
            pl.BlockSpec((2 * _D, _D), lambda i: (0, 0)),
            pl.BlockSpec((1, _D), lambda i: (0, 0)),
            pl.BlockSpec((1, 1, _BR), lambda i: (i, 0, 0)),
        ],
        out_specs=[
            pl.BlockSpec((_BR, _D), lambda i: (i, 0)),
            pl.BlockSpec((_G, _D), lambda i: (0, 0)),
        ],
        out_shape=[
            jax.ShapeDtypeStruct((_NP, _D), jnp.float32),
            jax.ShapeDtypeStruct((_G, _D), jnp.float32),
        ],
        scratch_shapes=[pltpu.VMEM((_G, _D), jnp.float32)],
        interpret=_INTERPRET,
    )(h2, seg, vn, vw1, vb1, vw2, vb2, batch3)


def _head_body(seg_ref, cnt_ref, pw_ref, pb_ref, out_ref):
    c = jnp.maximum(cnt_ref[0, :], 1.0)
    hg = seg_ref[...] * (1.0 / c)[:, None]
    out_ref[...] = jnp.dot(hg, pw_ref[...],
                           preferred_element_type=jnp.float32) + pb_ref[...]


def _head(seg, cnt, pw, pb1, t):
    return pl.pallas_call(
        _head_body,
        out_shape=jax.ShapeDtypeStruct((_G, t), jnp.float32),
        interpret=_INTERPRET,
    )(seg, cnt, pw, pb1)


def kernel(x, edge_index, batch, W1, b1, W2, b2, eps, gamma, beta,
           vW1, vb1, vW2, vb2, pW, pb):
    src3 = edge_index[0].reshape(_NW, _NCH, _C)
    dst3 = edge_index[1].reshape(_NW, _NCH, _C)
    zer = jnp.zeros((_RPT, _D), jnp.float32)
    t = pW.shape[1]
    batch_p = jnp.concatenate(
        [batch, jnp.full((_NP - _N,), _G, jnp.int32)])
    batch3 = batch_p.reshape(_NB, 1, _BR)
    xp = jnp.concatenate([x, jnp.zeros((_NP - _N, _D), jnp.float32)])
    eps1 = eps.reshape(_L, 1, 1)
    b1r = b1.reshape(_L, 1, 2 * _D)
    b2r = b2.reshape(_L, 1, _D)
    gmr = gamma.reshape(_L, 1, _D)
    btr = beta.reshape(_L, 1, _D)
    vb1r = vb1.reshape(_L - 1, 1, 2 * _D)
    vb2r = vb2.reshape(_L - 1, 1, _D)
    pb1 = pb.reshape(1, t)

    vn = jnp.zeros((_G, _D), jnp.float32)
    hp = xp
    seg = None
    cnt = None
    for l in range(_L):
        agg2 = _sc_agg(hp, src3, dst3, zer)
        last = l == _L - 1
        outs = _mlp_seg(hp, agg2, batch3, W1[l], b1r[l], W2[l], b2r[l],
                        gmr[l], btr[l], eps1[l], relu_out=not last,
                        with_cnt=last)
        if last:
            h2, seg, cnt = outs
        else:
            h2, seg = outs
            hp, vn = _vn_update(h2, seg, vn, vW1[l], vb1r[l], vW2[l],
                                vb2r[l], batch3)
    return _head(seg, cnt, pW, pb1, t)


# speedup vs baseline: 9.4386x; 1.3292x over previous
import functools

import jax
import jax.numpy as jnp
from jax import lax
from jax.experimental import pallas as pl
from jax.experimental.pallas import tpu as pltpu
from jax.experimental.pallas import tpu_sc as plsc

_N = 10000
_E = 320000
_D = 128
_G = 128
_L = 5
_NP = 10240
_BR = 512
_NB = _NP // _BR

_INTERPRET = False

_NC = 2
_NS = 16
_NW = _NC * _NS
_EPW = _E // _NW
_C = 100
_NCH = _EPW // _C
_RPT = _NP // _NS


def _sc_agg_body(hp_hbm, src_hbm, dst_hbm, zer_hbm, out_hbm,
                 sidx, didx, rows, acc_sh, isem, gsem):
    cid = lax.axis_index("c")
    sid = lax.axis_index("s")
    wid = sid * _NC + cid

    def start_idx(k, s):
        pltpu.async_copy(src_hbm.at[wid, k], sidx[s], isem[s])
        pltpu.async_copy(dst_hbm.at[wid, k], didx[s], isem[s])

    def wait_idx(k, s):
        pltpu.make_async_copy(src_hbm.at[wid, k], sidx[s], isem[s]).wait()
        pltpu.make_async_copy(dst_hbm.at[wid, k], didx[s], isem[s]).wait()

    def start_gather(s):
        pltpu.async_copy(hp_hbm.at[sidx[s].at[0]], rows[s], gsem[s])

    def wait_gather(s):
        pltpu.make_async_copy(hp_hbm.at[sidx[s].at[0]], rows[s],
                              gsem[s]).wait()

    def scatter(s):
        pltpu.sync_copy(rows[s], acc_sh.at[didx[s].at[0]], add=True)

    def visit(k, s0, s2):
        wait_gather(s0)

        @pl.when(k + 2 < _NCH)
        def _():
            wait_idx(k + 2, s2)
            start_gather(s2)

        scatter(s0)

        @pl.when(k + 3 < _NCH)
        def _():
            start_idx(k + 3, s0)

    start_idx(0, 0)
    start_idx(1, 1)
    start_idx(2, 2)
    pltpu.sync_copy(zer_hbm, acc_sh.at[pl.ds(sid * _RPT, _RPT)])
    plsc.subcore_barrier()
    wait_idx(0, 0)
    start_gather(0)
    wait_idx(1, 1)
    start_gather(1)
    visit(0, 0, 2)

    def body(b, carry):
        k = 3 * b + 1
        visit(k, 1, 0)
        visit(k + 1, 2, 1)
        visit(k + 2, 0, 2)
        return carry

    lax.fori_loop(0, (_NCH - 1) // 3, body, 0, unroll=False)
    plsc.subcore_barrier()
    pltpu.sync_copy(acc_sh.at[pl.ds(sid * _RPT, _RPT)],
                    out_hbm.at[cid, pl.ds(sid * _RPT, _RPT)])


def _sc_agg(hp, src4, dst4, zer):
    mesh = plsc.VectorSubcoreMesh(core_axis_name="c", subcore_axis_name="s",
                                  num_cores=_NC, num_subcores=_NS)
    f = pl.kernel(
        _sc_agg_body,
        out_type=jax.ShapeDtypeStruct((_NC, _NP, _D), jnp.float32),
        mesh=mesh,
        scratch_types=[
            [pltpu.VMEM((1, _C), jnp.int32) for _ in range(3)],
            [pltpu.VMEM((1, _C), jnp.int32) for _ in range(3)],
            [pltpu.VMEM((_C, _D), jnp.float32) for _ in range(3)],
            pltpu.VMEM_SHARED((_NP, _D), jnp.float32),
            [pltpu.SemaphoreType.DMA for _ in range(3)],
            [pltpu.SemaphoreType.DMA for _ in range(3)],
        ],
    )
    return f(hp, src4, dst4, zer)


def _bf(x):
    return x.astype(jnp.bfloat16)


def _mlp_seg_body(hp_ref, agg_ref, b3_ref, w1_ref, b1_ref, w2_ref, b2_ref,
                  gm_ref, bt_ref, eps_ref, *rest, relu_out, last):
    i = pl.program_id(0)
    e = eps_ref[0, 0]
    m = (1.0 + e) * hp_ref[...] + agg_ref[0] + agg_ref[1]
    a = jnp.maximum(
        jnp.dot(_bf(m), _bf(w1_ref[...]), preferred_element_type=jnp.float32)
        + b1_ref[...], 0.0)
    o = jnp.dot(_bf(a), _bf(w2_ref[...]),
                preferred_element_type=jnp.float32) + b2_ref[...]
    o = gm_ref[...] * o + bt_ref[...]
    if relu_out:
        o = jnp.maximum(o, 0.0)
    b = b3_ref[0, 0, :]
    oh = (b[:, None] == lax.broadcasted_iota(jnp.int32, (_BR, _G), 1)
          ).astype(jnp.float32)
    contrib = lax.dot_general(oh, o, (((0,), (0,)), ((), ())),
                              preferred_element_type=jnp.float32)
    if last:
        pw_ref, pb_ref, h2_ref, out_ref, seg_s, cnt_s = rest
    else:
        vn_ref, vw1_ref, vb1_ref, vw2_ref, vb2_ref, h2_ref, vno_ref, seg_s = \
            rest
    h2_ref[...] = o

    @pl.when(i == 0)
    def _():
        seg_s[...] = jnp.zeros_like(seg_s)

    seg_s[...] += contrib
    if last:
        @pl.when(i == 0)
        def _():
            cnt_s[...] = jnp.zeros_like(cnt_s)

        cnt_s[...] += jnp.sum(oh, axis=0, keepdims=True)

        @pl.when(i == _NB - 1)
        def _():
            c = jnp.maximum(cnt_s[0, :], 1.0)
            hg = seg_s[...] * (1.0 / c)[:, None]
            out_ref[...] = jnp.dot(
                hg, pw_ref[...], preferred_element_type=jnp.float32
            ) + pb_ref[...]
    else:
        @pl.when(i == _NB - 1)
        def _():
            vt = seg_s[...] + vn_ref[...]
            va = jnp.maximum(
                jnp.dot(vt, vw1_ref[...], preferred_element_type=jnp.float32)
                + vb1_ref[...], 0.0)
            vno_ref[...] = jnp.maximum(
                jnp.dot(va, vw2_ref[...], preferred_element_type=jnp.float32)
                + vb2_ref[...], 0.0)


def _full(shape):
    return pl.BlockSpec(shape, lambda i: tuple(0 for _ in shape))


def _mlp_seg(hp, agg2, batch3, w1, b1, w2, b2, gm, bt, eps1, extras,
             relu_out, last, t):
    base_specs = [
        pl.BlockSpec((_BR, _D), lambda i: (i, 0)),
        pl.BlockSpec((2, _BR, _D), lambda i: (0, i, 0)),
        pl.BlockSpec((1, 1, _BR), lambda i: (i, 0, 0)),
        _full((_D, 2 * _D)),
        _full((1, 2 * _D)),
        _full((2 * _D, _D)),
        _full((1, _D)),
        _full((1, _D)),
        _full((1, _D)),
        _full((1, 1)),
    ]
    if last:
        in_specs = base_specs + [_full((_D, t)), _full((1, t))]
        out_shapes = [jax.ShapeDtypeStruct((_NP, _D), jnp.float32),
                      jax.ShapeDtypeStruct((_G, t), jnp.float32)]
        out_specs = [pl.BlockSpec((_BR, _D), lambda i: (i, 0)),
                     _full((_G, t))]
        scratch = [pltpu.VMEM((_G, _D), jnp.float32),
                   pltpu.VMEM((1, _G), jnp.float32)]
    else:
        in_specs = base_specs + [_full((_G, _D)), _full((_D, 2 * _D)),
                                 _full((1, 2 * _D)), _full((2 * _D, _D)),
                                 _full((1, _D))]
        out_shapes = [jax.ShapeDtypeStruct((_NP, _D), jnp.float32),
                      jax.ShapeDtypeStruct((_G, _D), jnp.float32)]
        out_specs = [pl.BlockSpec((_BR, _D), lambda i: (i, 0)),
                     _full((_G, _D))]
        scratch = [pltpu.VMEM((_G, _D), jnp.float32)]
    return pl.pallas_call(
        functools.partial(_mlp_seg_body, relu_out=relu_out, last=last),
        grid=(_NB,),
        in_specs=in_specs,
        out_specs=out_specs,
        out_shape=out_shapes,
        scratch_shapes=scratch,
        interpret=_INTERPRET,
    )(hp, agg2, batch3, w1, b1, w2, b2, gm, bt, eps1, *extras)


def _layer_body(hp_ref, agg_ref, b3_ref, w1_ref, b1_ref, w2_ref, b2_ref,
                gm_ref, bt_ref, eps_ref, vn_ref, vw1_ref, vb1_ref, vw2_ref,
                vb2_ref, hpo_ref, vno_ref, h2_s, seg_s, vn_s):
    i = pl.program_id(0)

    @pl.when(i < _NB)
    def _():
        e = eps_ref[0, 0]
        m = (1.0 + e) * hp_ref[...] + agg_ref[0] + agg_ref[1]
        a = jnp.maximum(
            jnp.dot(_bf(m), _bf(w1_ref[...]),
                    preferred_element_type=jnp.float32) + b1_ref[...], 0.0)
        o = jnp.dot(_bf(a), _bf(w2_ref[...]),
                    preferred_element_type=jnp.float32) + b2_ref[...]
        o = jnp.maximum(gm_ref[...] * o + bt_ref[...], 0.0)
        h2_s[pl.ds(i * _BR, _BR), :] = o
        b = b3_ref[0, 0, :]
        oh = (b[:, None] == lax.broadcasted_iota(jnp.int32, (_BR, _G), 1)
              ).astype(jnp.float32)
        contrib = lax.dot_general(oh, o, (((0,), (0,)), ((), ())),
                                  preferred_element_type=jnp.float32)

        @pl.when(i == 0)
        def _():
            seg_s[...] = jnp.zeros_like(seg_s)

        seg_s[...] += contrib

        @pl.when(i == _NB - 1)
        def _():
            vt = seg_s[...] + vn_ref[...]
            va = jnp.maximum(
                jnp.dot(vt, vw1_ref[...], preferred_element_type=jnp.float32)
                + vb1_ref[...], 0.0)
            v2 = jnp.maximum(
                jnp.dot(va, vw2_ref[...], preferred_element_type=jnp.float32)
                + vb2_ref[...], 0.0)
            vn_s[...] = v2
            vno_ref[...] = v2

    @pl.when(i >= _NB)
    def _():
        b = b3_ref[0, 0, :]
        oh = (b[:, None] == lax.broadcasted_iota(jnp.int32, (_BR, _G), 1)
              ).astype(jnp.float32)
        hpo_ref[...] = h2_s[pl.ds((i - _NB) * _BR, _BR), :] + jnp.dot(
            oh, vn_s[...], preferred_element_type=jnp.float32)


def _layer(hp, agg2, batch3, w1, b1, w2, b2, gm, bt, eps1, vn,
           vw1, vb1, vw2, vb2):
    nb = _NB
    return pl.pallas_call(
        _layer_body,
        grid=(2 * nb,),
        in_specs=[
            pl.BlockSpec((_BR, _D), lambda i: (jnp.minimum(i, nb - 1), 0)),
            pl.BlockSpec((2, _BR, _D),
                         lambda i: (0, jnp.minimum(i, nb - 1), 0)),
            pl.BlockSpec((1, 1, _BR),
                         lambda i: (jnp.where(i < nb, i, i - nb), 0, 0)),
            _full((_D, 2 * _D)),
            _full((1, 2 * _D)),
            _full((2 * _D, _D)),
            _full((1, _D)),
            _full((1, _D)),
            _full((1, _D)),
            _full((1, 1)),
            _full((_G, _D)),
            _full((_D, 2 * _D)),
            _full((1, 2 * _D)),
            _full((2 * _D, _D)),
            _full((1, _D)),
        ],
        out_specs=[
            pl.BlockSpec((_BR, _D),
                         lambda i: (jnp.where(i < nb, 0, i - nb), 0)),
            _full((_G, _D)),
        ],
        out_shape=[
            jax.ShapeDtypeStruct((_NP, _D), jnp.float32),
            jax.ShapeDtypeStruct((_G, _D), jnp.float32),
        ],
        scratch_shapes=[
            pltpu.VMEM((_NP, _D), jnp.float32),
            pltpu.VMEM((_G, _D), jnp.float32),
            pltpu.VMEM((_G, _D), jnp.float32),
        ],
        interpret=_INTERPRET,
    )(hp, agg2, batch3, w1, b1, w2, b2, gm, bt, eps1, vn,
      vw1, vb1, vw2, vb2)


def kernel(x, edge_index, batch, W1, b1, W2, b2, eps, gamma, beta,
           vW1, vb1, vW2, vb2, pW, pb):
    src4 = edge_index[0].reshape(_NW, _NCH, 1, _C)
    dst4 = edge_index[1].reshape(_NW, _NCH, 1, _C)
    zer = jnp.zeros((_RPT, _D), jnp.float32)
    t = pW.shape[1]
    batch_p = jnp.concatenate(
        [batch, jnp.full((_NP - _N,), _G, jnp.int32)])
    batch3 = batch_p.reshape(_NB, 1, _BR)
    xp = jnp.concatenate([x, jnp.zeros((_NP - _N, _D), jnp.float32)])
    eps1 = eps.reshape(_L, 1, 1)
    b1r = b1.reshape(_L, 1, 2 * _D)
    b2r = b2.reshape(_L, 1, _D)
    gmr = gamma.reshape(_L, 1, _D)
    btr = beta.reshape(_L, 1, _D)
    vb1r = vb1.reshape(_L - 1, 1, 2 * _D)
    vb2r = vb2.reshape(_L - 1, 1, _D)
    pb1 = pb.reshape(1, t)

    vn = jnp.zeros((_G, _D), jnp.float32)
    hp = xp
    out = None
    for l in range(_L):
        agg2 = _sc_agg(hp, src4, dst4, zer)
        last = l == _L - 1
        if last:
            _, out = _mlp_seg(hp, agg2, batch3, W1[l], b1r[l], W2[l],
                              b2r[l], gmr[l], btr[l], eps1[l], [pW, pb1],
                              relu_out=False, last=True, t=t)
        else:
            hp, vn = _layer(hp, agg2, batch3, W1[l], b1r[l], W2[l], b2r[l],
                            gmr[l], btr[l], eps1[l], vn, vW1[l], vb1r[l],
                            vW2[l], vb2r[l])
    return out


# speedup vs baseline: 10.7420x; 1.1381x over previous
import jax
import jax.numpy as jnp
from jax import lax
from jax.experimental import pallas as pl
from jax.experimental.pallas import tpu as pltpu
from jax.experimental.pallas import tpu_sc as plsc

_N = 10000
_E = 320000
_D = 128
_G = 128
_L = 5
_NP = 10240
_BR = 512
_NB = _NP // _BR

_NC = 2
_NS = 16
_NW = _NC * _NS
_EPW = _E // _NW
_C = 125
_NCH = _EPW // _C
_RPT = _NP // _NS


def _sc_agg_body(hp_hbm, src_hbm, dst_hbm, zer_hbm, out_hbm,
                 sidx, didx, rows, acc_sh, isem, gsem):
    cid = lax.axis_index("c")
    sid = lax.axis_index("s")
    wid = sid * _NC + cid

    def start_idx(j, s):
        pltpu.async_copy(src_hbm.at[wid, j], sidx[s], isem[s])
        pltpu.async_copy(dst_hbm.at[wid, j], didx[s], isem[s])

    def wait_idx(j, s):
        pltpu.make_async_copy(src_hbm.at[wid, j], sidx[s], isem[s]).wait()
        pltpu.make_async_copy(dst_hbm.at[wid, j], didx[s], isem[s]).wait()

    def start_gather(s, half, r):
        pltpu.async_copy(hp_hbm.at[sidx[s].at[half]], rows[r], gsem[r])

    def scatter(s, half, r):
        pltpu.make_async_copy(hp_hbm.at[sidx[s].at[half]], rows[r],
                              gsem[r]).wait()
        pltpu.sync_copy(rows[r], acc_sh.at[didx[s].at[half]], add=True)

    start_idx(0, 0)
    start_idx(1, 1)
    pltpu.sync_copy(zer_hbm, acc_sh.at[pl.ds(sid * _RPT, _RPT)])
    plsc.subcore_barrier()
    wait_idx(0, 0)
    start_gather(0, 0, 0)

    nb4 = _NCH // 4

    def body(b, carry):
        j0 = 2 * b
        more = b < nb4 - 1
        start_gather(0, 1, 1)
        scatter(0, 0, 0)
        wait_idx(j0 + 1, 1)
        start_gather(1, 0, 0)
        scatter(0, 1, 1)

        @pl.when(more)
        def _():
            start_idx(j0 + 2, 0)

        start_gather(1, 1, 1)
        scatter(1, 0, 0)

        @pl.when(more)
        def _():
            wait_idx(j0 + 2, 0)
            start_gather(0, 0, 0)

        scatter(1, 1, 1)

        @pl.when(more)
        def _():
            start_idx(j0 + 3, 1)

        return carry

    lax.fori_loop(0, nb4, body, 0, unroll=False)
    plsc.subcore_barrier()
    pltpu.sync_copy(acc_sh.at[pl.ds(sid * _RPT, _RPT)],
                    out_hbm.at[cid, pl.ds(sid * _RPT, _RPT)])


def _sc_agg(hp, src4, dst4, zer):
    mesh = plsc.VectorSubcoreMesh(core_axis_name="c", subcore_axis_name="s",
                                  num_cores=_NC, num_subcores=_NS)
    f = pl.kernel(
        _sc_agg_body,
        out_type=jax.ShapeDtypeStruct((_NC, _NP, _D), jnp.float32),
        mesh=mesh,
        scratch_types=[
            [pltpu.VMEM((2, _C), jnp.int32) for _ in range(2)],
            [pltpu.VMEM((2, _C), jnp.int32) for _ in range(2)],
            [pltpu.VMEM((_C, _D), jnp.float32) for _ in range(2)],
            pltpu.VMEM_SHARED((_NP, _D), jnp.float32),
            [pltpu.SemaphoreType.DMA for _ in range(2)],
            [pltpu.SemaphoreType.DMA for _ in range(2)],
        ],
    )
    return f(hp, src4, dst4, zer)


def _bf(x):
    return x.astype(jnp.bfloat16)


def _mlp_seg_body(hp_ref, agg_ref, b3_ref, w1_ref, b1_ref, w2_ref, b2_ref,
                  gm_ref, bt_ref, eps_ref, pw_ref, pb_ref, h2_ref, out_ref,
                  seg_s, cnt_s):
    i = pl.program_id(0)
    e = eps_ref[0, 0]
    m = (1.0 + e) * hp_ref[...] + agg_ref[0] + agg_ref[1]
    a = jnp.maximum(
        jnp.dot(_bf(m), _bf(w1_ref[...]), preferred_element_type=jnp.float32)
        + b1_ref[...], 0.0)
    o = jnp.dot(_bf(a), _bf(w2_ref[...]),
                preferred_element_type=jnp.float32) + b2_ref[...]
    o = gm_ref[...] * o + bt_ref[...]
    b = b3_ref[0, 0, :]
    oh = (b[:, None] == lax.broadcasted_iota(jnp.int32, (_BR, _G), 1)
          ).astype(jnp.float32)
    contrib = lax.dot_general(oh, o, (((0,), (0,)), ((), ())),
                              preferred_element_type=jnp.float32)
    h2_ref[...] = o

    @pl.when(i == 0)
    def _():
        seg_s[...] = jnp.zeros_like(seg_s)
        cnt_s[...] = jnp.zeros_like(cnt_s)

    seg_s[...] += contrib
    cnt_s[...] += jnp.sum(oh, axis=0, keepdims=True)

    @pl.when(i == _NB - 1)
    def _():
        c = jnp.maximum(cnt_s[0, :], 1.0)
        hg = seg_s[...] * (1.0 / c)[:, None]
        out_ref[...] = jnp.dot(
            hg, pw_ref[...], preferred_element_type=jnp.float32
        ) + pb_ref[...]


def _full(shape):
    return pl.BlockSpec(shape, lambda i: tuple(0 for _ in shape))


def _mlp_seg(hp, agg2, batch3, w1, b1, w2, b2, gm, bt, eps1, pw, pb1, t):
    in_specs = [
        pl.BlockSpec((_BR, _D), lambda i: (i, 0)),
        pl.BlockSpec((2, _BR, _D), lambda i: (0, i, 0)),
        pl.BlockSpec((1, 1, _BR), lambda i: (i, 0, 0)),
        _full((_D, 2 * _D)),
        _full((1, 2 * _D)),
        _full((2 * _D, _D)),
        _full((1, _D)),
        _full((1, _D)),
        _full((1, _D)),
        _full((1, 1)),
        _full((_D, t)),
        _full((1, t)),
    ]
    return pl.pallas_call(
        _mlp_seg_body,
        grid=(_NB,),
        in_specs=in_specs,
        out_specs=[pl.BlockSpec((_BR, _D), lambda i: (i, 0)),
                   _full((_G, t))],
        out_shape=[jax.ShapeDtypeStruct((_NP, _D), jnp.float32),
                   jax.ShapeDtypeStruct((_G, t), jnp.float32)],
        scratch_shapes=[pltpu.VMEM((_G, _D), jnp.float32),
                        pltpu.VMEM((1, _G), jnp.float32)],
    )(hp, agg2, batch3, w1, b1, w2, b2, gm, bt, eps1, pw, pb1)


def _layer_body(hp_ref, agg_ref, b3_ref, w1_ref, b1_ref, w2_ref, b2_ref,
                gm_ref, bt_ref, eps_ref, vn_ref, vw1_ref, vb1_ref, vw2_ref,
                vb2_ref, hpo_ref, vno_ref, h2_s, seg_s, vn_s):
    i = pl.program_id(0)

    @pl.when(i < _NB)
    def _():
        e = eps_ref[0, 0]
        m = (1.0 + e) * hp_ref[...] + agg_ref[0] + agg_ref[1]
        a = jnp.maximum(
            jnp.dot(_bf(m), _bf(w1_ref[...]),
                    preferred_element_type=jnp.float32) + b1_ref[...], 0.0)
        o = jnp.dot(_bf(a), _bf(w2_ref[...]),
                    preferred_element_type=jnp.float32) + b2_ref[...]
        o = jnp.maximum(gm_ref[...] * o + bt_ref[...], 0.0)
        h2_s[pl.ds(i * _BR, _BR), :] = o
        b = b3_ref[0, 0, :]
        oh = (b[:, None] == lax.broadcasted_iota(jnp.int32, (_BR, _G), 1)
              ).astype(jnp.float32)
        contrib = lax.dot_general(oh, o, (((0,), (0,)), ((), ())),
                                  preferred_element_type=jnp.float32)

        @pl.when(i == 0)
        def _():
            seg_s[...] = jnp.zeros_like(seg_s)

        seg_s[...] += contrib

        @pl.when(i == _NB - 1)
        def _():
            vt = seg_s[...] + vn_ref[...]
            va = jnp.maximum(
                jnp.dot(vt, vw1_ref[...], preferred_element_type=jnp.float32)
                + vb1_ref[...], 0.0)
            v2 = jnp.maximum(
                jnp.dot(va, vw2_ref[...], preferred_element_type=jnp.float32)
                + vb2_ref[...], 0.0)
            vn_s[...] = v2
            vno_ref[...] = v2

    @pl.when(i >= _NB)
    def _():
        b = b3_ref[0, 0, :]
        oh = (b[:, None] == lax.broadcasted_iota(jnp.int32, (_BR, _G), 1)
              ).astype(jnp.float32)
        hpo_ref[...] = h2_s[pl.ds((i - _NB) * _BR, _BR), :] + jnp.dot(
            oh, vn_s[...], preferred_element_type=jnp.float32)


def _layer(hp, agg2, batch3, w1, b1, w2, b2, gm, bt, eps1, vn,
           vw1, vb1, vw2, vb2):
    nb = _NB
    return pl.pallas_call(
        _layer_body,
        grid=(2 * nb,),
        in_specs=[
            pl.BlockSpec((_BR, _D), lambda i: (jnp.minimum(i, nb - 1), 0)),
            pl.BlockSpec((2, _BR, _D),
                         lambda i: (0, jnp.minimum(i, nb - 1), 0)),
            pl.BlockSpec((1, 1, _BR),
                         lambda i: (jnp.where(i < nb, i, i - nb), 0, 0)),
            _full((_D, 2 * _D)),
            _full((1, 2 * _D)),
            _full((2 * _D, _D)),
            _full((1, _D)),
            _full((1, _D)),
            _full((1, _D)),
            _full((1, 1)),
            _full((_G, _D)),
            _full((_D, 2 * _D)),
            _full((1, 2 * _D)),
            _full((2 * _D, _D)),
            _full((1, _D)),
        ],
        out_specs=[
            pl.BlockSpec((_BR, _D),
                         lambda i: (jnp.where(i < nb, 0, i - nb), 0)),
            _full((_G, _D)),
        ],
        out_shape=[
            jax.ShapeDtypeStruct((_NP, _D), jnp.float32),
            jax.ShapeDtypeStruct((_G, _D), jnp.float32),
        ],
        scratch_shapes=[
            pltpu.VMEM((_NP, _D), jnp.float32),
            pltpu.VMEM((_G, _D), jnp.float32),
            pltpu.VMEM((_G, _D), jnp.float32),
        ],
    )(hp, agg2, batch3, w1, b1, w2, b2, gm, bt, eps1, vn,
      vw1, vb1, vw2, vb2)


def kernel(x, edge_index, batch, W1, b1, W2, b2, eps, gamma, beta,
           vW1, vb1, vW2, vb2, pW, pb):
    src4 = edge_index[0].reshape(_NW, _NCH // 2, 2, _C)
    dst4 = edge_index[1].reshape(_NW, _NCH // 2, 2, _C)
    zer = jnp.zeros((_RPT, _D), jnp.float32)
    t = pW.shape[1]
    batch_p = jnp.concatenate(
        [batch, jnp.full((_NP - _N,), _G, jnp.int32)])
    batch3 = batch_p.reshape(_NB, 1, _BR)
    xp = jnp.concatenate([x, jnp.zeros((_NP - _N, _D), jnp.float32)])
    eps1 = eps.reshape(_L, 1, 1)
    b1r = b1.reshape(_L, 1, 2 * _D)
    b2r = b2.reshape(_L, 1, _D)
    gmr = gamma.reshape(_L, 1, _D)
    btr = beta.reshape(_L, 1, _D)
    vb1r = vb1.reshape(_L - 1, 1, 2 * _D)
    vb2r = vb2.reshape(_L - 1, 1, _D)
    pb1 = pb.reshape(1, t)

    vn = jnp.zeros((_G, _D), jnp.float32)
    hp = xp
    out = None
    for l in range(_L):
        agg2 = _sc_agg(hp, src4, dst4, zer)
        last = l == _L - 1
        if last:
            _, out = _mlp_seg(hp, agg2, batch3, W1[l], b1r[l], W2[l],
                              b2r[l], gmr[l], btr[l], eps1[l], pW, pb1, t)
        else:
            hp, vn = _layer(hp, agg2, batch3, W1[l], b1r[l], W2[l], b2r[l],
                            gmr[l], btr[l], eps1[l], vn, vW1[l], vb1r[l],
                            vW2[l], vb2r[l])
    return out
